# baseline (speedup 1.0000x reference)
import functools

import jax
import jax.numpy as jnp
from jax.experimental import pallas as pl
from jax.experimental.pallas import tpu as pltpu

ALPHA = 5.0
EPS = 1e-8


def _cosine_loss_kernel(labels_ref, x_ref, out_ref, *, n_rows, n_cols,
                        block_rows):
    rb = pl.program_id(0)

    @pl.when(rb == 0)
    def _init():
        out_ref[0, 0] = 0.0

    x = x_ref[...]
    ss = jnp.sum(x * x, axis=1, keepdims=True)
    lcol = jax.lax.broadcasted_iota(jnp.int32, (block_rows, n_cols), 1)
    g = jnp.sum(jnp.where(lcol == labels_ref[...], x, 0.0), axis=1,
                keepdims=True)
    s = g / jnp.maximum(jnp.sqrt(ss), EPS)
    loss_terms = (1.0 - s) / (1.0 + s) * ALPHA
    out_ref[0, 0] += jnp.sum(loss_terms) / n_rows


def kernel(logits, labels):
    n_rows, n_cols = logits.shape
    block_rows = 16
    n_blocks = n_rows // block_rows
    labels2 = labels.astype(jnp.int32).reshape(n_rows, 1)

    out = pl.pallas_call(
        functools.partial(
            _cosine_loss_kernel, n_rows=n_rows, n_cols=n_cols,
            block_rows=block_rows),
        grid=(n_blocks,),
        in_specs=[
            pl.BlockSpec((block_rows, 1), lambda rb: (rb, 0)),
            pl.BlockSpec((block_rows, n_cols), lambda rb: (rb, 0)),
        ],
        out_specs=pl.BlockSpec(
            (1, 1), lambda rb: (0, 0), memory_space=pltpu.SMEM),
        out_shape=jax.ShapeDtypeStruct((1, 1), jnp.float32),
    )(labels2, logits)
    return out[0, 0]


# speedup vs baseline: 1.0601x; 1.0601x over previous
import functools

import jax
import jax.numpy as jnp
from jax.experimental import pallas as pl
from jax.experimental.pallas import tpu as pltpu

ALPHA = 5.0
EPS = 1e-8


def _cosine_loss_kernel(labels_ref, x_ref, out_ref, *, n_rows, n_cols,
                        block_rows):
    rb = pl.program_id(0)

    @pl.when(rb == 0)
    def _init():
        out_ref[0, 0] = 0.0

    x = x_ref[...]
    ss = jnp.sum(x * x, axis=1, keepdims=True)
    lcol = jax.lax.broadcasted_iota(jnp.int32, (block_rows, n_cols), 1)
    g = jnp.sum(jnp.where(lcol == labels_ref[...], x, 0.0), axis=1,
                keepdims=True)
    s = g / jnp.maximum(jnp.sqrt(ss), EPS)
    loss_terms = (1.0 - s) / (1.0 + s) * ALPHA
    out_ref[0, 0] += jnp.sum(loss_terms) / n_rows


def kernel(logits, labels):
    n_rows, n_cols = logits.shape
    block_rows = 32
    n_blocks = n_rows // block_rows
    labels2 = labels.astype(jnp.int32).reshape(n_rows, 1)

    out = pl.pallas_call(
        functools.partial(
            _cosine_loss_kernel, n_rows=n_rows, n_cols=n_cols,
            block_rows=block_rows),
        grid=(n_blocks,),
        in_specs=[
            pl.BlockSpec((block_rows, 1), lambda rb: (rb, 0)),
            pl.BlockSpec((block_rows, n_cols), lambda rb: (rb, 0)),
        ],
        out_specs=pl.BlockSpec(
            (1, 1), lambda rb: (0, 0), memory_space=pltpu.SMEM),
        out_shape=jax.ShapeDtypeStruct((1, 1), jnp.float32),
    )(labels2, logits)
    return out[0, 0]
